# baseline (speedup 1.0000x reference)
import functools

import jax
import jax.numpy as jnp
from jax import lax
from jax.experimental import pallas as pl
from jax.experimental.pallas import tpu as pltpu
from jax.experimental.pallas import tpu_sc as plsc

VOCAB = 100000
DIM = 64
B = 4096
L = 50
N = B * L

NUM_CORES = 2
NUM_SUBCORES = 16
LANES = 16
NW = NUM_CORES * NUM_SUBCORES
PAIRS_PER_WORKER = N // NW
CHUNK = 128
CHUNKS_PER_WORKER = PAIRS_PER_WORKER // CHUNK
GROUPS = CHUNK // LANES
STRIDE17 = LANES + 1
QSTRIDE = STRIDE17 * LANES
PADL = 128
ROWS_PER_WORKER = B // NW

_EPS2 = 1e-24


def _rsqrt(x):
    i = plsc.bitcast(x, jnp.int32)
    y = plsc.bitcast(jnp.int32(0x5F3759DF) - (i >> 1), jnp.float32)
    for _ in range(3):
        y = y * (1.5 - 0.5 * x * y * y)
    return y


def _body(idx1_hbm, idx2_hbm, table_hbm, out_hbm,
          idx1_v, idx2_v, ic1a, ic2a, ic1b, ic2b, r1a, r2a, r1b, r2b, out_v,
          stage, sem_a, sem_b):
    wid = lax.axis_index("s") * NUM_CORES + lax.axis_index("c")
    base = wid * PAIRS_PER_WORKER

    pltpu.sync_copy(idx1_hbm.at[pl.ds(wid * ROWS_PER_WORKER,
                                      ROWS_PER_WORKER)], idx1_v)
    pltpu.sync_copy(idx2_hbm.at[pl.ds(wid * ROWS_PER_WORKER,
                                      ROWS_PER_WORKER)], idx2_v)

    lane = lax.iota(jnp.int32, LANES)

    def compact(c, ic1, ic2):
        for g in range(GROUPS):
            q = c * CHUNK + g * LANES + lane
            lrow = q // L
            col = q - lrow * L
            ic1[pl.ds(g * LANES, LANES)] = plsc.load_gather(idx1_v, [lrow, col])
            ic2[pl.ds(g * LANES, LANES)] = plsc.load_gather(idx2_v, [lrow, col])

    def start(c, ic1, ic2, d1, d2, sem):
        compact(c, ic1, ic2)
        pltpu.async_copy(table_hbm.at[ic1], d1, sem)
        pltpu.async_copy(table_hbm.at[ic2], d2, sem)

    def wait(c, ic1, ic2, d1, d2, sem):
        pltpu.make_async_copy(table_hbm.at[ic1], d1, sem).wait()
        pltpu.make_async_copy(table_hbm.at[ic2], d2, sem).wait()

    lane17 = lane * STRIDE17

    def colsum(qbase):
        idx = lane17 + qbase
        y = plsc.load_gather(stage, [idx])
        for _ in range(1, LANES):
            idx = idx + 1
            y = y + plsc.load_gather(stage, [idx])
        return y

    def compute(c, d1, d2):
        @plsc.parallel_loop(0, GROUPS, unroll=2)
        def group_body(g):
            sbase = g * (3 * QSTRIDE)
            for u in range(LANES):
                p = g * LANES + u
                acc_d = None
                acc_1 = None
                acc_2 = None
                for k in range(DIM // LANES):
                    a = d1[p, pl.ds(k * LANES, LANES)]
                    b = d2[p, pl.ds(k * LANES, LANES)]
                    if acc_d is None:
                        acc_d, acc_1, acc_2 = a * b, a * a, b * b
                    else:
                        acc_d += a * b
                        acc_1 += a * a
                        acc_2 += b * b
                stage[pl.ds(sbase + u * STRIDE17, LANES)] = acc_d
                stage[pl.ds(sbase + QSTRIDE + u * STRIDE17, LANES)] = acc_1
                stage[pl.ds(sbase + 2 * QSTRIDE + u * STRIDE17, LANES)] = acc_2
            vd = colsum(sbase)
            v1 = jnp.maximum(colsum(sbase + QSTRIDE), _EPS2)
            v2 = jnp.maximum(colsum(sbase + 2 * QSTRIDE), _EPS2)
            cos = vd * _rsqrt(v1) * _rsqrt(v2)
            q = c * CHUNK + g * LANES + lane
            lrow = q // L
            plsc.store_scatter(out_v, [lrow, q - lrow * L], cos)

    start(0, ic1a, ic2a, r1a, r2a, sem_a)
    start(1, ic1b, ic2b, r1b, r2b, sem_b)

    def loop_body(cc, carry):
        c0 = 2 * cc
        wait(c0, ic1a, ic2a, r1a, r2a, sem_a)
        compute(c0, r1a, r2a)

        @pl.when(cc < CHUNKS_PER_WORKER // 2 - 1)
        def _():
            start(c0 + 2, ic1a, ic2a, r1a, r2a, sem_a)

        wait(c0 + 1, ic1b, ic2b, r1b, r2b, sem_b)
        compute(c0 + 1, r1b, r2b)

        @pl.when(cc < CHUNKS_PER_WORKER // 2 - 1)
        def _():
            start(c0 + 3, ic1b, ic2b, r1b, r2b, sem_b)

        return carry

    lax.fori_loop(0, CHUNKS_PER_WORKER // 2, loop_body, jnp.int32(0))

    pltpu.sync_copy(out_v, out_hbm.at[pl.ds(wid * ROWS_PER_WORKER,
                                            ROWS_PER_WORKER)])


@functools.partial(
    pl.kernel,
    out_type=jax.ShapeDtypeStruct((B, PADL), jnp.float32),
    name="sc_cosine",
    mesh=plsc.VectorSubcoreMesh(core_axis_name="c", subcore_axis_name="s"),
    compiler_params=pltpu.CompilerParams(
        needs_layout_passes=False, use_tc_tiling_on_sc=False
    ),
    scratch_types=[
        pltpu.VMEM((ROWS_PER_WORKER, PADL), jnp.int32),
        pltpu.VMEM((ROWS_PER_WORKER, PADL), jnp.int32),
        pltpu.VMEM((CHUNK,), jnp.int32),
        pltpu.VMEM((CHUNK,), jnp.int32),
        pltpu.VMEM((CHUNK,), jnp.int32),
        pltpu.VMEM((CHUNK,), jnp.int32),
        pltpu.VMEM((CHUNK, DIM), jnp.float32),
        pltpu.VMEM((CHUNK, DIM), jnp.float32),
        pltpu.VMEM((CHUNK, DIM), jnp.float32),
        pltpu.VMEM((CHUNK, DIM), jnp.float32),
        pltpu.VMEM((ROWS_PER_WORKER, PADL), jnp.float32),
        pltpu.VMEM((GROUPS * 3 * QSTRIDE,), jnp.float32),
        pltpu.SemaphoreType.DMA,
        pltpu.SemaphoreType.DMA,
    ],
)
def _sc_cosine(idx1_hbm, idx2_hbm, table_hbm, out_hbm, *scratch):
    _body(idx1_hbm, idx2_hbm, table_hbm, out_hbm, *scratch)


_PAD_BLK = 1024


def _pad_body(i_ref, o_ref):
    o_ref[...] = jnp.concatenate(
        [i_ref[...], jnp.zeros((_PAD_BLK, PADL - L), jnp.int32)], axis=1
    )


_pad = pl.pallas_call(
    _pad_body,
    grid=(B // _PAD_BLK,),
    in_specs=[pl.BlockSpec((_PAD_BLK, L), lambda i: (i, 0))],
    out_specs=pl.BlockSpec((_PAD_BLK, PADL), lambda i: (i, 0)),
    out_shape=jax.ShapeDtypeStruct((B, PADL), jnp.int32),
)


def kernel(idx1, idx2, table):
    out = _sc_cosine(_pad(idx1), _pad(idx2), table)
    return out[:, :L]


# speedup vs baseline: 1.0357x; 1.0357x over previous
import functools

import jax
import jax.numpy as jnp
from jax import lax
from jax.experimental import pallas as pl
from jax.experimental.pallas import tpu as pltpu
from jax.experimental.pallas import tpu_sc as plsc

VOCAB = 100000
DIM = 64
B = 4096
L = 50
N = B * L

NUM_CORES = 2
NUM_SUBCORES = 16
LANES = 16
NW = NUM_CORES * NUM_SUBCORES
PAIRS_PER_WORKER = N // NW
CHUNK = 128
CHUNKS_PER_WORKER = PAIRS_PER_WORKER // CHUNK
GROUPS = CHUNK // LANES
STRIDE17 = LANES + 1
QSTRIDE = STRIDE17 * LANES

_EPS2 = 1e-24


def _rsqrt(x):
    i = plsc.bitcast(x, jnp.int32)
    y = plsc.bitcast(jnp.int32(0x5F3759DF) - (i >> 1), jnp.float32)
    for _ in range(3):
        y = y * (1.5 - 0.5 * x * y * y)
    return y


def _body(idx1_hbm, idx2_hbm, table_hbm, out_hbm,
          idx1_v, idx2_v, r1a, r2a, r1b, r2b, out_v,
          stage, sem_a, sem_b):
    wid = lax.axis_index("s") * NUM_CORES + lax.axis_index("c")
    base = wid * PAIRS_PER_WORKER

    pltpu.sync_copy(idx1_hbm.at[pl.ds(base, PAIRS_PER_WORKER)], idx1_v)
    pltpu.sync_copy(idx2_hbm.at[pl.ds(base, PAIRS_PER_WORKER)], idx2_v)

    lane = lax.iota(jnp.int32, LANES)

    def start(c, d1, d2, sem):
        i1 = idx1_v.at[pl.ds(c * CHUNK, CHUNK)]
        i2 = idx2_v.at[pl.ds(c * CHUNK, CHUNK)]
        pltpu.async_copy(table_hbm.at[i1], d1, sem)
        pltpu.async_copy(table_hbm.at[i2], d2, sem)

    def wait(c, d1, d2, sem):
        i1 = idx1_v.at[pl.ds(c * CHUNK, CHUNK)]
        i2 = idx2_v.at[pl.ds(c * CHUNK, CHUNK)]
        pltpu.make_async_copy(table_hbm.at[i1], d1, sem).wait()
        pltpu.make_async_copy(table_hbm.at[i2], d2, sem).wait()

    lane17 = lane * STRIDE17

    def colsum(qbase):
        idx = lane17 + qbase
        y = plsc.load_gather(stage, [idx])
        for _ in range(1, LANES):
            idx = idx + 1
            y = y + plsc.load_gather(stage, [idx])
        return y

    def compute(c, d1, d2):
        @plsc.parallel_loop(0, GROUPS, unroll=2)
        def group_body(g):
            sbase = g * (3 * QSTRIDE)
            for u in range(LANES):
                p = g * LANES + u
                acc_d = None
                acc_1 = None
                acc_2 = None
                for k in range(DIM // LANES):
                    a = d1[p, pl.ds(k * LANES, LANES)]
                    b = d2[p, pl.ds(k * LANES, LANES)]
                    if acc_d is None:
                        acc_d, acc_1, acc_2 = a * b, a * a, b * b
                    else:
                        acc_d += a * b
                        acc_1 += a * a
                        acc_2 += b * b
                stage[pl.ds(sbase + u * STRIDE17, LANES)] = acc_d
                stage[pl.ds(sbase + QSTRIDE + u * STRIDE17, LANES)] = acc_1
                stage[pl.ds(sbase + 2 * QSTRIDE + u * STRIDE17, LANES)] = acc_2
            vd = colsum(sbase)
            v1 = jnp.maximum(colsum(sbase + QSTRIDE), _EPS2)
            v2 = jnp.maximum(colsum(sbase + 2 * QSTRIDE), _EPS2)
            cos = vd * _rsqrt(v1) * _rsqrt(v2)
            out_v[pl.ds(c * CHUNK + g * LANES, LANES)] = cos

    start(0, r1a, r2a, sem_a)
    start(1, r1b, r2b, sem_b)

    def loop_body(cc, carry):
        c0 = 2 * cc
        wait(c0, r1a, r2a, sem_a)
        compute(c0, r1a, r2a)

        @pl.when(cc < CHUNKS_PER_WORKER // 2 - 1)
        def _():
            start(c0 + 2, r1a, r2a, sem_a)

        wait(c0 + 1, r1b, r2b, sem_b)
        compute(c0 + 1, r1b, r2b)

        @pl.when(cc < CHUNKS_PER_WORKER // 2 - 1)
        def _():
            start(c0 + 3, r1b, r2b, sem_b)

        return carry

    lax.fori_loop(0, CHUNKS_PER_WORKER // 2, loop_body, jnp.int32(0))

    pltpu.sync_copy(out_v, out_hbm.at[pl.ds(base, PAIRS_PER_WORKER)])


@functools.partial(
    pl.kernel,
    out_type=jax.ShapeDtypeStruct((N,), jnp.float32),
    mesh=plsc.VectorSubcoreMesh(core_axis_name="c", subcore_axis_name="s"),
    compiler_params=pltpu.CompilerParams(
        needs_layout_passes=False, use_tc_tiling_on_sc=False
    ),
    scratch_types=[
        pltpu.VMEM((PAIRS_PER_WORKER,), jnp.int32),
        pltpu.VMEM((PAIRS_PER_WORKER,), jnp.int32),
        pltpu.VMEM((CHUNK, DIM), jnp.float32),
        pltpu.VMEM((CHUNK, DIM), jnp.float32),
        pltpu.VMEM((CHUNK, DIM), jnp.float32),
        pltpu.VMEM((CHUNK, DIM), jnp.float32),
        pltpu.VMEM((PAIRS_PER_WORKER,), jnp.float32),
        pltpu.VMEM((GROUPS * 3 * QSTRIDE,), jnp.float32),
        pltpu.SemaphoreType.DMA,
        pltpu.SemaphoreType.DMA,
    ],
)
def _sc_cosine(idx1_hbm, idx2_hbm, table_hbm, out_hbm, *scratch):
    _body(idx1_hbm, idx2_hbm, table_hbm, out_hbm, *scratch)


def kernel(idx1, idx2, table):
    out = _sc_cosine(idx1.reshape(N), idx2.reshape(N), table)
    return out.reshape(B, L)
